# baseline (speedup 1.0000x reference)
import jax
import jax.numpy as jnp
from jax.experimental import pallas as pl
from jax.experimental.pallas import tpu as pltpu

N_EDGES = 320000
N_NODES = 10000
D = 128

REDUCE_BLOCK = 4000
NODE_BLOCK = 2000


def _reduce_body(x_ref, out_ref):
    step = pl.program_id(0)

    @pl.when(step == 0)
    def _():
        out_ref[...] = jnp.zeros_like(out_ref)

    x = x_ref[...]
    partial = jnp.sum(x.reshape(REDUCE_BLOCK // 8, 8, D), axis=0)
    out_ref[...] += partial


def _fused_body(src_ref, w_self_ref, w_aggr_ref, sums_ref, out_ref):
    self_hidden = jnp.dot(src_ref[...], w_self_ref[...],
                          preferred_element_type=jnp.float32)
    mean = jnp.sum(sums_ref[...], axis=0, keepdims=True) * (1.0 / N_EDGES)
    nh = jnp.dot(mean, w_aggr_ref[...], preferred_element_type=jnp.float32)
    out_ref[:, :D] = jnp.maximum(self_hidden, 0.0)
    out_ref[:, D:] = jnp.broadcast_to(jnp.maximum(nh, 0.0),
                                      (out_ref.shape[0], D))


def kernel(src_node_features, neighbor_node_features, W_aggr, W_self):
    sums = pl.pallas_call(
        _reduce_body,
        grid=(N_EDGES // REDUCE_BLOCK,),
        in_specs=[pl.BlockSpec((REDUCE_BLOCK, D), lambda i: (i, 0))],
        out_specs=pl.BlockSpec((8, D), lambda i: (0, 0)),
        out_shape=jax.ShapeDtypeStruct((8, D), jnp.float32),
    )(neighbor_node_features)

    out = pl.pallas_call(
        _fused_body,
        grid=(N_NODES // NODE_BLOCK,),
        in_specs=[
            pl.BlockSpec((NODE_BLOCK, D), lambda i: (i, 0)),
            pl.BlockSpec((D, D), lambda i: (0, 0)),
            pl.BlockSpec((D, D), lambda i: (0, 0)),
            pl.BlockSpec((8, D), lambda i: (0, 0)),
        ],
        out_specs=pl.BlockSpec((NODE_BLOCK, 2 * D), lambda i: (i, 0)),
        out_shape=jax.ShapeDtypeStruct((N_NODES, 2 * D), jnp.float32),
    )(src_node_features, W_self, W_aggr, sums)
    return out


# speedup vs baseline: 1.3851x; 1.3851x over previous
import jax
import jax.numpy as jnp
from jax.experimental import pallas as pl
from jax.experimental.pallas import tpu as pltpu

N_EDGES = 320000
N_NODES = 10000
D = 128

REDUCE_BLOCK = 6400
ACC_ROWS = 256
NODE_BLOCK = 2000


def _reduce_body(x_ref, out_ref):
    step = pl.program_id(0)

    @pl.when(step == 0)
    def _():
        out_ref[...] = jnp.zeros_like(out_ref)

    x = x_ref[...]
    partial = jnp.sum(x.reshape(REDUCE_BLOCK // ACC_ROWS, ACC_ROWS, D), axis=0)
    out_ref[...] += partial


def _fused_body(src_ref, w_self_ref, w_aggr_ref, sums_ref, out_ref):
    self_hidden = jnp.dot(src_ref[...], w_self_ref[...],
                          preferred_element_type=jnp.float32)
    mean = jnp.sum(sums_ref[...], axis=0, keepdims=True) * (1.0 / N_EDGES)
    nh = jnp.dot(mean, w_aggr_ref[...], preferred_element_type=jnp.float32)
    out_ref[:, :D] = jnp.maximum(self_hidden, 0.0)
    out_ref[:, D:] = jnp.broadcast_to(jnp.maximum(nh, 0.0),
                                      (out_ref.shape[0], D))


def kernel(src_node_features, neighbor_node_features, W_aggr, W_self):
    sums = pl.pallas_call(
        _reduce_body,
        grid=(N_EDGES // REDUCE_BLOCK,),
        in_specs=[pl.BlockSpec((REDUCE_BLOCK, D), lambda i: (i, 0))],
        out_specs=pl.BlockSpec((ACC_ROWS, D), lambda i: (0, 0)),
        out_shape=jax.ShapeDtypeStruct((ACC_ROWS, D), jnp.float32),
    )(neighbor_node_features)

    out = pl.pallas_call(
        _fused_body,
        grid=(N_NODES // NODE_BLOCK,),
        in_specs=[
            pl.BlockSpec((NODE_BLOCK, D), lambda i: (i, 0)),
            pl.BlockSpec((D, D), lambda i: (0, 0)),
            pl.BlockSpec((D, D), lambda i: (0, 0)),
            pl.BlockSpec((ACC_ROWS, D), lambda i: (0, 0)),
        ],
        out_specs=pl.BlockSpec((NODE_BLOCK, 2 * D), lambda i: (i, 0)),
        out_shape=jax.ShapeDtypeStruct((N_NODES, 2 * D), jnp.float32),
    )(src_node_features, W_self, W_aggr, sums)
    return out
